# Initial kernel scaffold; baseline (speedup 1.0000x reference)
import jax
import jax.numpy as jnp
from jax.experimental import pallas as pl


def kernel(x, Wq, Wk, Wv, gate_logits, Wk_c, Wv_c, Wi_q, Wi_k, Wi_g, sink_logit, Wo):
    raise NotImplementedError("write your pallas kernel here")


# speedup vs baseline: 10.1809x; 10.1809x over previous
import functools
import math

import jax
import jax.numpy as jnp
from jax.experimental import pallas as pl

D = 1024
L = 2048
H = 16
HD = 64
NIH = 4
HDI = 256
LC = 511
LCP = 512
NSEL = 64
WIN = 128
QB = 256
NQB = L // QB
SCALE = math.sqrt(HD)
NEG_BF = float(jnp.finfo(jnp.bfloat16).min)
IMIN = -2147483647 - 1
IMAX = 2147483647


def _compress_kernel(x4a_ref, x4b_ref, g_ref, wik_ref, wkc_ref, wvc_ref,
                     ki_ref, kc_ref, vc_ref):
    g = g_ref[...]
    g = g - jnp.max(g, axis=1, keepdims=True)
    e = jnp.exp(g)
    gw = e / jnp.sum(e, axis=1, keepdims=True)
    xa = x4a_ref[...]
    xb = x4b_ref[...]
    acc = jnp.zeros((LCP, D), jnp.float32)
    for r in range(4):
        acc = acc + gw[0, r] * xa[:, r * D:(r + 1) * D]
    for r in range(4):
        acc = acc + gw[0, 4 + r] * xb[:, r * D:(r + 1) * D]
    row = jax.lax.broadcasted_iota(jnp.int32, (LCP, D), 0)
    xc = jnp.where(row < LC, acc, 0.0)
    ki_ref[...] = jnp.dot(xc, wik_ref[...],
                          preferred_element_type=jnp.float32)
    xcb = xc.astype(jnp.bfloat16)
    kc_ref[...] = jnp.dot(xcb, wkc_ref[...],
                          preferred_element_type=jnp.float32
                          ).astype(jnp.bfloat16)
    vc_ref[...] = jnp.dot(xcb, wvc_ref[...],
                          preferred_element_type=jnp.float32
                          ).astype(jnp.bfloat16)


def _scores_kernel(x_ref, wiq_ref, wig_ref, ki_ref, bias_ref):
    x = x_ref[...]
    qi = jnp.dot(x, wiq_ref[...], preferred_element_type=jnp.float32)
    wg = jnp.dot(x, wig_ref[...], preferred_element_type=jnp.float32)
    ki = ki_ref[...]
    s = jnp.zeros((QB, LCP), jnp.float32)
    for h in range(NIH):
        qk = jax.lax.dot_general(
            qi[:, h * HDI:(h + 1) * HDI], ki[:, h * HDI:(h + 1) * HDI],
            (((1,), (1,)), ((), ())), preferred_element_type=jnp.float32)
        s = s + jnp.maximum(qk, 0.0) * wg[:, h:h + 1]
    col = jax.lax.broadcasted_iota(jnp.int32, (QB, LCP), 1)
    s = jnp.where(col < LC, s, NEG_BF)
    bits = jax.lax.bitcast_convert_type(s, jnp.int32)
    keys = bits ^ (jax.lax.shift_right_arithmetic(bits, 31) & IMAX)
    cpos = jnp.sum((keys >= 0).astype(jnp.int32), axis=1, keepdims=True)
    p0 = cpos >= NSEL
    lo = jnp.where(p0, 0, IMIN).astype(jnp.int32)
    hi = jnp.where(p0, IMAX, -1).astype(jnp.int32)

    def body(_, carry):
        lo, hi = carry
        d = hi - lo
        mid = lo + jax.lax.shift_right_arithmetic(d, 1) + (d & 1)
        cnt = jnp.sum((keys >= mid).astype(jnp.int32), axis=1,
                      keepdims=True)
        p = cnt >= NSEL
        return jnp.where(p, mid, lo), jnp.where(p, hi, mid - 1)

    lo, hi = jax.lax.fori_loop(0, 31, body, (lo, hi))
    gt = keys > lo
    tie = keys == lo
    need = NSEL - jnp.sum(gt.astype(jnp.int32), axis=1, keepdims=True)
    lo2 = jnp.zeros_like(need)
    hi2 = jnp.full_like(need, LCP - 1)

    def body2(_, carry):
        lo2, hi2 = carry
        mid = jax.lax.shift_right_arithmetic(lo2 + hi2, 1)
        cnt = jnp.sum((tie & (col <= mid)).astype(jnp.int32), axis=1,
                      keepdims=True)
        p = cnt >= need
        return jnp.where(p, lo2, mid + 1), jnp.where(p, mid, hi2)

    lo2, hi2 = jax.lax.fori_loop(0, 9, body2, (lo2, hi2))
    sel = gt | (tie & (col <= lo2))
    bias_ref[...] = jnp.where(sel, 0.0, NEG_BF).astype(jnp.bfloat16)


def _proj_kernel(xb_ref, wq_ref, wk_ref, wv_ref, psw_ref, c_ref, s_ref,
                 q_ref, k_ref, v_ref):
    xb = xb_ref[...]
    cos = c_ref[...]
    sin = s_ref[...]
    psw = psw_ref[...]

    def rope(w_ref):
        a = jnp.dot(xb, w_ref[...], preferred_element_type=jnp.float32)
        ab = a.astype(jnp.bfloat16)
        asw = jnp.dot(ab, psw, preferred_element_type=jnp.float32)
        return (ab.astype(jnp.float32) * cos + asw * sin
                ).astype(jnp.bfloat16)

    q_ref[...] = rope(wq_ref)
    k_ref[...] = rope(wk_ref)
    v_ref[...] = jnp.dot(xb, wv_ref[...],
                         preferred_element_type=jnp.float32
                         ).astype(jnp.bfloat16)


def _attn_kernel(q_ref, kp_ref, kc_ref, vp_ref, vc_ref, kcc_ref, vcc_ref,
                 bias_ref, sink_ref, o_ref):
    i = pl.program_id(0)
    q = q_ref[0]
    cs = jax.lax.dot_general(q, kcc_ref[0], (((1,), (1,)), ((), ())),
                             preferred_element_type=jnp.float32)
    cs = (cs / SCALE).astype(jnp.bfloat16).astype(jnp.float32)
    cs = cs + bias_ref[...].astype(jnp.float32)
    k2 = jnp.concatenate([kp_ref[0], kc_ref[0]], axis=0)
    ws = jax.lax.dot_general(q, k2, (((1,), (1,)), ((), ())),
                             preferred_element_type=jnp.float32)
    ws = (ws / SCALE).astype(jnp.bfloat16).astype(jnp.float32)
    row = i * QB + jax.lax.broadcasted_iota(jnp.int32, (QB, 2 * QB), 0)
    colb = jnp.where(i > 0, (i - 1) * QB, -QB)
    col = colb + jax.lax.broadcasted_iota(jnp.int32, (QB, 2 * QB), 1)
    valid = (col >= 0) & (col <= row) & (row - col < WIN)
    ws = jnp.where(valid, ws, NEG_BF)
    sink = sink_ref[0, 0, 0]
    m = jnp.maximum(jnp.maximum(jnp.max(cs, axis=1, keepdims=True),
                                jnp.max(ws, axis=1, keepdims=True)), sink)
    ec = jnp.exp(cs - m)
    ew = jnp.exp(ws - m)
    es = jnp.exp(sink - m)
    den = (jnp.sum(ec, axis=1, keepdims=True)
           + jnp.sum(ew, axis=1, keepdims=True) + es)
    wc = (ec / den).astype(jnp.bfloat16)
    ww = (ew / den).astype(jnp.bfloat16)
    v2 = jnp.concatenate([vp_ref[0], vc_ref[0]], axis=0)
    oc = jnp.dot(wc, vcc_ref[0],
                 preferred_element_type=jnp.float32).astype(jnp.bfloat16)
    ow = jnp.dot(ww, v2,
                 preferred_element_type=jnp.float32).astype(jnp.bfloat16)
    o_ref[0] = oc + ow


def _wo_kernel(a_ref, wo_ref, o_ref):
    acc = jnp.zeros((QB, D), jnp.float32)
    for h in range(H):
        acc += jnp.dot(a_ref[h], wo_ref[h],
                       preferred_element_type=jnp.float32)
    o_ref[...] = acc.astype(jnp.bfloat16)


@jax.jit
def kernel(x, Wq, Wk, Wv, gate_logits, Wk_c, Wv_c, Wi_q, Wi_k, Wi_g,
           sink_logit, Wo):
    bf = jnp.bfloat16
    x2 = x[0]
    x4a = x2.reshape(LCP, 4 * D)
    x4b = jnp.concatenate([x2[4:], jnp.zeros((4, D), x2.dtype)]
                          ).reshape(LCP, 4 * D)
    g2 = gate_logits.reshape(1, 8)

    half = HD // 2
    inv_freq = 1.0 / (10000.0 ** (jnp.arange(half, dtype=jnp.float32)
                                  / half))
    t = jnp.arange(L, dtype=jnp.float32)
    fr = jnp.outer(t, inv_freq)
    cos = jnp.cos(fr)
    sin = jnp.sin(fr)
    ctab = jnp.tile(jnp.concatenate([cos, cos], axis=1), (1, H))
    stab = jnp.tile(jnp.concatenate([-sin, sin], axis=1), (1, H))
    perm = jnp.arange(D) ^ 32
    psw = jnp.eye(D, dtype=bf)[perm]

    full = lambda shp: pl.BlockSpec(shp, lambda *_: (0, 0))

    ki, kc, vc = pl.pallas_call(
        _compress_kernel,
        out_shape=(jax.ShapeDtypeStruct((LCP, D), jnp.float32),
                   jax.ShapeDtypeStruct((LCP, D), bf),
                   jax.ShapeDtypeStruct((LCP, D), bf)),
        in_specs=[full((LCP, 4 * D)), full((LCP, 4 * D)), full((1, 8)),
                  full((D, D)), full((D, D)), full((D, D))],
        out_specs=(full((LCP, D)), full((LCP, D)), full((LCP, D))),
    )(x4a, x4b, g2, Wi_k, Wk_c.astype(bf), Wv_c.astype(bf))

    bias = pl.pallas_call(
        _scores_kernel,
        grid=(NQB,),
        out_shape=jax.ShapeDtypeStruct((L, LCP), bf),
        in_specs=[pl.BlockSpec((QB, D), lambda i: (i, 0)),
                  pl.BlockSpec((D, D), lambda i: (0, 0)),
                  pl.BlockSpec((D, NIH), lambda i: (0, 0)),
                  pl.BlockSpec((LCP, D), lambda i: (0, 0))],
        out_specs=pl.BlockSpec((QB, LCP), lambda i: (i, 0)),
    )(x2, Wi_q, Wi_g, ki)

    xb16 = x2.astype(bf)
    q, k, v = pl.pallas_call(
        _proj_kernel,
        grid=(NQB,),
        out_shape=(jax.ShapeDtypeStruct((L, D), bf),) * 3,
        in_specs=[pl.BlockSpec((QB, D), lambda i: (i, 0))]
        + [pl.BlockSpec((D, D), lambda i: (0, 0))] * 4
        + [pl.BlockSpec((QB, D), lambda i: (i, 0))] * 2,
        out_specs=(pl.BlockSpec((QB, D), lambda i: (i, 0)),) * 3,
    )(xb16, Wq.astype(bf), Wk.astype(bf), Wv.astype(bf), psw, ctab, stab)

    q3 = q.reshape(L, H, HD).transpose(1, 0, 2)
    k3 = k.reshape(L, H, HD).transpose(1, 0, 2)
    v3 = v.reshape(L, H, HD).transpose(1, 0, 2)
    kc3 = kc.reshape(LCP, H, HD).transpose(1, 0, 2)
    vc3 = vc.reshape(LCP, H, HD).transpose(1, 0, 2)
    sink3 = sink_logit.reshape(H, 1, 1)

    attn = pl.pallas_call(
        _attn_kernel,
        grid=(NQB, H),
        out_shape=jax.ShapeDtypeStruct((H, L, HD), bf),
        in_specs=[
            pl.BlockSpec((1, QB, HD), lambda i, h: (h, i, 0)),
            pl.BlockSpec((1, QB, HD),
                         lambda i, h: (h, jnp.maximum(i - 1, 0), 0)),
            pl.BlockSpec((1, QB, HD), lambda i, h: (h, i, 0)),
            pl.BlockSpec((1, QB, HD),
                         lambda i, h: (h, jnp.maximum(i - 1, 0), 0)),
            pl.BlockSpec((1, QB, HD), lambda i, h: (h, i, 0)),
            pl.BlockSpec((1, LCP, HD), lambda i, h: (h, 0, 0)),
            pl.BlockSpec((1, LCP, HD), lambda i, h: (h, 0, 0)),
            pl.BlockSpec((QB, LCP), lambda i, h: (i, 0)),
            pl.BlockSpec((1, 1, 1), lambda i, h: (h, 0, 0)),
        ],
        out_specs=pl.BlockSpec((1, QB, HD), lambda i, h: (h, i, 0)),
    )(q3, k3, k3, v3, v3, kc3, vc3, bias, sink3)

    out = pl.pallas_call(
        _wo_kernel,
        grid=(NQB,),
        out_shape=jax.ShapeDtypeStruct((L, D), bf),
        in_specs=[pl.BlockSpec((H, QB, HD), lambda i: (0, i, 0)),
                  pl.BlockSpec((H, HD, D), lambda i: (0, 0, 0))],
        out_specs=pl.BlockSpec((QB, D), lambda i: (i, 0)),
    )(attn, Wo.astype(bf).reshape(H, HD, D))
    return out[None]


# speedup vs baseline: 12.5558x; 1.2333x over previous
import functools
import math

import jax
import jax.numpy as jnp
from jax.experimental import pallas as pl
from jax.experimental.pallas import tpu as pltpu

D = 1024
L = 2048
H = 16
HD = 64
NIH = 4
HDI = 256
LC = 511
LCP = 512
NSEL = 64
WIN = 128
QB = 256
NQB = L // QB
SCALE = math.sqrt(HD)
NEG_BF = float(jnp.finfo(jnp.bfloat16).min)
IMIN = -2147483647 - 1
IMAX = 2147483647


def _compress_kernel(x4a_ref, x4b_ref, g_ref, wik_ref, wkc_ref, wvc_ref,
                     ki_ref, kc_ref, vc_ref):
    g = g_ref[...]
    g = g - jnp.max(g, axis=1, keepdims=True)
    e = jnp.exp(g)
    gw = e / jnp.sum(e, axis=1, keepdims=True)
    xa = x4a_ref[...]
    xb = x4b_ref[...]
    acc = jnp.zeros((LCP, D), jnp.float32)
    for r in range(4):
        acc = acc + gw[0, r] * xa[:, r * D:(r + 1) * D]
    for r in range(4):
        acc = acc + gw[0, 4 + r] * xb[:, r * D:(r + 1) * D]
    row = jax.lax.broadcasted_iota(jnp.int32, (LCP, D), 0)
    xc = jnp.where(row < LC, acc, 0.0)
    ki_ref[...] = jnp.dot(xc, wik_ref[...],
                          preferred_element_type=jnp.float32)
    xcb = xc.astype(jnp.bfloat16)
    kcf = jnp.dot(xcb, wkc_ref[...],
                  preferred_element_type=jnp.float32).astype(jnp.bfloat16)
    vcf = jnp.dot(xcb, wvc_ref[...],
                  preferred_element_type=jnp.float32).astype(jnp.bfloat16)
    for h in range(H):
        kc_ref[h] = kcf[:, h * HD:(h + 1) * HD]
        vc_ref[h] = vcf[:, h * HD:(h + 1) * HD]


def _scores_kernel(x_ref, wiq_ref, wig_ref, ki_ref, bias_ref):
    x = x_ref[...]
    qi = jnp.dot(x, wiq_ref[...], preferred_element_type=jnp.float32)
    wg = jnp.dot(x, wig_ref[...], preferred_element_type=jnp.float32)
    ki = ki_ref[...]
    s = jnp.zeros((QB, LCP), jnp.float32)
    for h in range(NIH):
        qk = jax.lax.dot_general(
            qi[:, h * HDI:(h + 1) * HDI], ki[:, h * HDI:(h + 1) * HDI],
            (((1,), (1,)), ((), ())), preferred_element_type=jnp.float32)
        s = s + jnp.maximum(qk, 0.0) * wg[:, h:h + 1]
    col = jax.lax.broadcasted_iota(jnp.int32, (QB, LCP), 1)
    s = jnp.where(col < LC, s, NEG_BF)
    bits = jax.lax.bitcast_convert_type(s, jnp.int32)
    keys = bits ^ (jax.lax.shift_right_arithmetic(bits, 31) & IMAX)
    cpos = jnp.sum((keys >= 0).astype(jnp.int32), axis=1, keepdims=True)
    p0 = cpos >= NSEL
    lo = jnp.where(p0, 0, IMIN).astype(jnp.int32)
    hi = jnp.where(p0, IMAX, -1).astype(jnp.int32)

    def body(_, carry):
        lo, hi = carry
        d = hi - lo
        mid = lo + jax.lax.shift_right_arithmetic(d, 1) + (d & 1)
        cnt = jnp.sum((keys >= mid).astype(jnp.int32), axis=1,
                      keepdims=True)
        p = cnt >= NSEL
        return jnp.where(p, mid, lo), jnp.where(p, hi, mid - 1)

    lo, hi = jax.lax.fori_loop(0, 31, body, (lo, hi))
    gt = keys > lo
    tie = keys == lo
    need = NSEL - jnp.sum(gt.astype(jnp.int32), axis=1, keepdims=True)
    lo2 = jnp.zeros_like(need)
    hi2 = jnp.full_like(need, LCP - 1)

    def body2(_, carry):
        lo2, hi2 = carry
        mid = jax.lax.shift_right_arithmetic(lo2 + hi2, 1)
        cnt = jnp.sum((tie & (col <= mid)).astype(jnp.int32), axis=1,
                      keepdims=True)
        p = cnt >= need
        return jnp.where(p, lo2, mid + 1), jnp.where(p, mid, hi2)

    lo2, hi2 = jax.lax.fori_loop(0, 9, body2, (lo2, hi2))
    sel = gt | (tie & (col <= lo2))
    bias_ref[...] = jnp.where(sel, 0.0, NEG_BF).astype(jnp.bfloat16)


def _fused_kernel(xb_ref, wq_ref, wk_ref, wv_ref, psw_ref, c_ref, s_ref,
                  kc_ref, vc_ref, bias_ref, sink_ref, wo_ref, o_ref,
                  kprev_ref, vprev_ref):
    i = pl.program_id(0)
    xb = xb_ref[...]
    cos = c_ref[...]
    sin = s_ref[...]
    psw = psw_ref[...]

    def rope(w_ref):
        a = jnp.dot(xb, w_ref[...], preferred_element_type=jnp.float32)
        ab = a.astype(jnp.bfloat16)
        asw = jnp.dot(ab, psw, preferred_element_type=jnp.float32)
        return (ab.astype(jnp.float32) * cos + asw * sin
                ).astype(jnp.bfloat16)

    q = rope(wq_ref)
    k = rope(wk_ref)
    v = jnp.dot(xb, wv_ref[...],
                preferred_element_type=jnp.float32).astype(jnp.bfloat16)

    @pl.when(i == 0)
    def _():
        kprev_ref[...] = jnp.zeros_like(kprev_ref)
        vprev_ref[...] = jnp.zeros_like(vprev_ref)

    kp = kprev_ref[...]
    vp = vprev_ref[...]
    biasf = bias_ref[...].astype(jnp.float32)
    row = i * QB + jax.lax.broadcasted_iota(jnp.int32, (QB, 2 * QB), 0)
    col = (i - 1) * QB + jax.lax.broadcasted_iota(jnp.int32, (QB, 2 * QB), 1)
    valid = (col >= 0) & (col <= row) & (row - col < WIN)

    acc = jnp.zeros((QB, D), jnp.float32)
    for h in range(H):
        sl = slice(h * HD, (h + 1) * HD)
        qh = q[:, sl]
        cs = jax.lax.dot_general(qh, kc_ref[h], (((1,), (1,)), ((), ())),
                                 preferred_element_type=jnp.float32)
        cs = (cs / SCALE).astype(jnp.bfloat16).astype(jnp.float32) + biasf
        k2 = jnp.concatenate([kp[:, sl], k[:, sl]], axis=0)
        ws = jax.lax.dot_general(qh, k2, (((1,), (1,)), ((), ())),
                                 preferred_element_type=jnp.float32)
        ws = (ws / SCALE).astype(jnp.bfloat16).astype(jnp.float32)
        ws = jnp.where(valid, ws, NEG_BF)
        sink = sink_ref[h, 0, 0]
        m = jnp.maximum(jnp.maximum(jnp.max(cs, axis=1, keepdims=True),
                                    jnp.max(ws, axis=1, keepdims=True)),
                        sink)
        ec = jnp.exp(cs - m)
        ew = jnp.exp(ws - m)
        es = jnp.exp(sink - m)
        den = (jnp.sum(ec, axis=1, keepdims=True)
               + jnp.sum(ew, axis=1, keepdims=True) + es)
        wc = (ec / den).astype(jnp.bfloat16)
        ww = (ew / den).astype(jnp.bfloat16)
        v2 = jnp.concatenate([vp[:, sl], v[:, sl]], axis=0)
        oc = jnp.dot(wc, vc_ref[h],
                     preferred_element_type=jnp.float32).astype(jnp.bfloat16)
        ow = jnp.dot(ww, v2,
                     preferred_element_type=jnp.float32).astype(jnp.bfloat16)
        acc = acc + jnp.dot(oc + ow, wo_ref[h],
                            preferred_element_type=jnp.float32)
    o_ref[...] = acc.astype(jnp.bfloat16)
    kprev_ref[...] = k
    vprev_ref[...] = v


@jax.jit
def kernel(x, Wq, Wk, Wv, gate_logits, Wk_c, Wv_c, Wi_q, Wi_k, Wi_g,
           sink_logit, Wo):
    bf = jnp.bfloat16
    x2 = x[0]
    x4a = x2.reshape(LCP, 4 * D)
    x4b = jnp.concatenate([x2[4:], jnp.zeros((4, D), x2.dtype)]
                          ).reshape(LCP, 4 * D)
    g2 = gate_logits.reshape(1, 8)

    half = HD // 2
    inv_freq = 1.0 / (10000.0 ** (jnp.arange(half, dtype=jnp.float32)
                                  / half))
    t = jnp.arange(L, dtype=jnp.float32)
    fr = jnp.outer(t, inv_freq)
    cos = jnp.cos(fr)
    sin = jnp.sin(fr)
    ctab = jnp.tile(jnp.concatenate([cos, cos], axis=1), (1, H))
    stab = jnp.tile(jnp.concatenate([-sin, sin], axis=1), (1, H))
    perm = jnp.arange(D) ^ 32
    psw = jnp.eye(D, dtype=bf)[perm]

    full = lambda shp: pl.BlockSpec(shp, lambda *_: (0, 0))

    full3 = lambda shp: pl.BlockSpec(shp, lambda *_: (0, 0, 0))

    ki, kc3, vc3 = pl.pallas_call(
        _compress_kernel,
        out_shape=(jax.ShapeDtypeStruct((LCP, D), jnp.float32),
                   jax.ShapeDtypeStruct((H, LCP, HD), bf),
                   jax.ShapeDtypeStruct((H, LCP, HD), bf)),
        in_specs=[full((LCP, 4 * D)), full((LCP, 4 * D)), full((1, 8)),
                  full((D, D)), full((D, D)), full((D, D))],
        out_specs=(full((LCP, D)), full3((H, LCP, HD)),
                   full3((H, LCP, HD))),
    )(x4a, x4b, g2, Wi_k, Wk_c.astype(bf), Wv_c.astype(bf))

    bias = pl.pallas_call(
        _scores_kernel,
        grid=(NQB,),
        out_shape=jax.ShapeDtypeStruct((L, LCP), bf),
        in_specs=[pl.BlockSpec((QB, D), lambda i: (i, 0)),
                  pl.BlockSpec((D, D), lambda i: (0, 0)),
                  pl.BlockSpec((D, NIH), lambda i: (0, 0)),
                  pl.BlockSpec((LCP, D), lambda i: (0, 0))],
        out_specs=pl.BlockSpec((QB, LCP), lambda i: (i, 0)),
    )(x2, Wi_q, Wi_g, ki)

    xb16 = x2.astype(bf)
    sink3 = sink_logit.reshape(H, 1, 1)
    out = pl.pallas_call(
        _fused_kernel,
        grid=(NQB,),
        out_shape=jax.ShapeDtypeStruct((L, D), bf),
        in_specs=[
            pl.BlockSpec((QB, D), lambda i: (i, 0)),
            pl.BlockSpec((D, D), lambda i: (0, 0)),
            pl.BlockSpec((D, D), lambda i: (0, 0)),
            pl.BlockSpec((D, D), lambda i: (0, 0)),
            pl.BlockSpec((D, D), lambda i: (0, 0)),
            pl.BlockSpec((QB, D), lambda i: (i, 0)),
            pl.BlockSpec((QB, D), lambda i: (i, 0)),
            pl.BlockSpec((H, LCP, HD), lambda i: (0, 0, 0)),
            pl.BlockSpec((H, LCP, HD), lambda i: (0, 0, 0)),
            pl.BlockSpec((QB, LCP), lambda i: (i, 0)),
            pl.BlockSpec((H, 1, 1), lambda i: (0, 0, 0)),
            pl.BlockSpec((H, HD, D), lambda i: (0, 0, 0)),
        ],
        out_specs=pl.BlockSpec((QB, D), lambda i: (i, 0)),
        scratch_shapes=[pltpu.VMEM((QB, D), bf),
                        pltpu.VMEM((QB, D), bf)],
    )(xb16, Wq.astype(bf), Wk.astype(bf), Wv.astype(bf), psw, ctab, stab,
      kc3, vc3, bias, sink3, Wo.astype(bf).reshape(H, HD, D))
    return out[None]


# speedup vs baseline: 14.2727x; 1.1367x over previous
import functools
import math

import jax
import jax.numpy as jnp
from jax.experimental import pallas as pl
from jax.experimental.pallas import tpu as pltpu

D = 1024
L = 2048
H = 16
HD = 64
NIH = 4
HDI = 256
LC = 511
LCP = 512
NSEL = 64
WIN = 128
QB = 256
NQB = L // QB
SCALE = math.sqrt(HD)
NEG_BF = float(jnp.finfo(jnp.bfloat16).min)
IMIN = -2147483647 - 1
IMAX = 2147483647


def _compress_kernel(x4a_ref, x4b_ref, g_ref, wik_ref, wkc_ref, wvc_ref,
                     ki_ref, kc_ref, vc_ref):
    g = g_ref[...]
    g = g - jnp.max(g, axis=1, keepdims=True)
    e = jnp.exp(g)
    gw = e / jnp.sum(e, axis=1, keepdims=True)
    xa = x4a_ref[...]
    xb = x4b_ref[...]
    acc = jnp.zeros((LCP, D), jnp.float32)
    for r in range(4):
        acc = acc + gw[0, r] * xa[:, r * D:(r + 1) * D]
    for r in range(4):
        acc = acc + gw[0, 4 + r] * xb[:, r * D:(r + 1) * D]
    row = jax.lax.broadcasted_iota(jnp.int32, (LCP, D), 0)
    xc = jnp.where(row < LC, acc, 0.0)
    ki_ref[...] = jnp.dot(xc, wik_ref[...],
                          preferred_element_type=jnp.float32)
    xcb = xc.astype(jnp.bfloat16)
    kcf = jnp.dot(xcb, wkc_ref[...],
                  preferred_element_type=jnp.float32).astype(jnp.bfloat16)
    vcf = jnp.dot(xcb, wvc_ref[...],
                  preferred_element_type=jnp.float32).astype(jnp.bfloat16)
    for h in range(H):
        kc_ref[h] = kcf[:, h * HD:(h + 1) * HD]
        vc_ref[h] = vcf[:, h * HD:(h + 1) * HD]


def _scores_kernel(x_ref, wiq_ref, wig_ref, ki_ref, bias_ref):
    x = x_ref[...]
    qi = jnp.dot(x, wiq_ref[...], preferred_element_type=jnp.float32)
    wg = jnp.dot(x, wig_ref[...], preferred_element_type=jnp.float32)
    ki = ki_ref[...]
    s = jnp.zeros((QB, LCP), jnp.float32)
    for h in range(NIH):
        qk = jax.lax.dot_general(
            qi[:, h * HDI:(h + 1) * HDI], ki[:, h * HDI:(h + 1) * HDI],
            (((1,), (1,)), ((), ())), preferred_element_type=jnp.float32)
        s = s + jnp.maximum(qk, 0.0) * wg[:, h:h + 1]
    col = jax.lax.broadcasted_iota(jnp.int32, (QB, LCP), 1)
    s = jnp.where(col < LC, s, NEG_BF)
    bits = jax.lax.bitcast_convert_type(s, jnp.int32)
    keys = bits ^ (jax.lax.shift_right_arithmetic(bits, 31) & IMAX)
    cpos = jnp.sum((keys >= 0).astype(jnp.int32), axis=1, keepdims=True)
    p0 = cpos >= NSEL
    lo = jnp.where(p0, 0, IMIN).astype(jnp.int32)
    hi = jnp.where(p0, IMAX, -1).astype(jnp.int32)

    def body(_, carry):
        lo, hi = carry
        d = hi - lo
        mid = lo + jax.lax.shift_right_arithmetic(d, 1) + (d & 1)
        cnt = jnp.sum((keys >= mid).astype(jnp.int32), axis=1,
                      keepdims=True)
        p = cnt >= NSEL
        return jnp.where(p, mid, lo), jnp.where(p, hi, mid - 1)

    lo, hi = jax.lax.fori_loop(0, 31, body, (lo, hi))
    gt = keys > lo
    tie = keys == lo
    need = NSEL - jnp.sum(gt.astype(jnp.int32), axis=1, keepdims=True)
    lo2 = jnp.zeros_like(need)
    hi2 = jnp.full_like(need, LCP - 1)

    def body2(_, carry):
        lo2, hi2 = carry
        mid = jax.lax.shift_right_arithmetic(lo2 + hi2, 1)
        cnt = jnp.sum((tie & (col <= mid)).astype(jnp.int32), axis=1,
                      keepdims=True)
        p = cnt >= need
        return jnp.where(p, lo2, mid + 1), jnp.where(p, mid, hi2)

    lo2, hi2 = jax.lax.fori_loop(0, 9, body2, (lo2, hi2))
    sel = gt | (tie & (col <= lo2))
    bias_ref[...] = jnp.where(sel, 0.0, NEG_BF).astype(jnp.bfloat16)


def _fused_kernel(xb_ref, wq_ref, wk_ref, wv_ref, c_ref, s_ref,
                  kc_ref, vc_ref, bias_ref, sink_ref, wo_ref, o_ref,
                  kprev_ref, vprev_ref):
    i = pl.program_id(0)
    xb = xb_ref[...]
    c32 = c_ref[...]
    s32 = s_ref[...]
    cos = jnp.concatenate([jnp.concatenate([c32, c32], axis=1)] * H,
                          axis=1)
    sin = jnp.concatenate([jnp.concatenate([-s32, s32], axis=1)] * H,
                          axis=1)

    def rope(w_ref):
        a = jnp.dot(xb, w_ref[...], preferred_element_type=jnp.float32)
        ab = a.astype(jnp.bfloat16)
        asw = jnp.concatenate(
            [p for h in range(H)
             for p in (ab[:, h * HD + 32:(h + 1) * HD],
                       ab[:, h * HD:h * HD + 32])], axis=1)
        return (ab.astype(jnp.float32) * cos
                + asw.astype(jnp.float32) * sin).astype(jnp.bfloat16)

    q = rope(wq_ref)
    k = rope(wk_ref)
    v = jnp.dot(xb, wv_ref[...],
                preferred_element_type=jnp.float32).astype(jnp.bfloat16)

    @pl.when(i == 0)
    def _():
        kprev_ref[...] = jnp.zeros_like(kprev_ref)
        vprev_ref[...] = jnp.zeros_like(vprev_ref)

    kp = kprev_ref[...]
    vp = vprev_ref[...]
    biasf = bias_ref[...].astype(jnp.float32)
    row = i * QB + jax.lax.broadcasted_iota(jnp.int32, (QB, 2 * QB), 0)
    col = (i - 1) * QB + jax.lax.broadcasted_iota(jnp.int32, (QB, 2 * QB), 1)
    valid = (col >= 0) & (col <= row) & (row - col < WIN)

    acc = jnp.zeros((QB, D), jnp.float32)
    for h in range(H):
        sl = slice(h * HD, (h + 1) * HD)
        qh = q[:, sl]
        cs = jax.lax.dot_general(qh, kc_ref[h], (((1,), (1,)), ((), ())),
                                 preferred_element_type=jnp.float32)
        cs = (cs / SCALE).astype(jnp.bfloat16).astype(jnp.float32) + biasf
        k2 = jnp.concatenate([kp[:, sl], k[:, sl]], axis=0)
        ws = jax.lax.dot_general(qh, k2, (((1,), (1,)), ((), ())),
                                 preferred_element_type=jnp.float32)
        ws = (ws / SCALE).astype(jnp.bfloat16).astype(jnp.float32)
        ws = jnp.where(valid, ws, NEG_BF)
        sink = sink_ref[h, 0, 0]
        m = jnp.maximum(jnp.maximum(jnp.max(cs, axis=1, keepdims=True),
                                    jnp.max(ws, axis=1, keepdims=True)),
                        sink)
        ec = jnp.exp(cs - m)
        ew = jnp.exp(ws - m)
        es = jnp.exp(sink - m)
        den = (jnp.sum(ec, axis=1, keepdims=True)
               + jnp.sum(ew, axis=1, keepdims=True) + es)
        r = 1.0 / den
        wc = (ec * r).astype(jnp.bfloat16)
        ww = (ew * r).astype(jnp.bfloat16)
        v2 = jnp.concatenate([vp[:, sl], v[:, sl]], axis=0)
        oc = jnp.dot(wc, vc_ref[h],
                     preferred_element_type=jnp.float32).astype(jnp.bfloat16)
        ow = jnp.dot(ww, v2,
                     preferred_element_type=jnp.float32).astype(jnp.bfloat16)
        acc = acc + jnp.dot(oc + ow, wo_ref[h],
                            preferred_element_type=jnp.float32)
    o_ref[...] = acc.astype(jnp.bfloat16)
    kprev_ref[...] = k
    vprev_ref[...] = v


@jax.jit
def kernel(x, Wq, Wk, Wv, gate_logits, Wk_c, Wv_c, Wi_q, Wi_k, Wi_g,
           sink_logit, Wo):
    bf = jnp.bfloat16
    x2 = x[0]
    x4a = x2.reshape(LCP, 4 * D)
    x4b = jnp.concatenate([x2[4:], jnp.zeros((4, D), x2.dtype)]
                          ).reshape(LCP, 4 * D)
    g2 = gate_logits.reshape(1, 8)

    half = HD // 2
    inv_freq = 1.0 / (10000.0 ** (jnp.arange(half, dtype=jnp.float32)
                                  / half))
    t = jnp.arange(L, dtype=jnp.float32)
    fr = jnp.outer(t, inv_freq)
    ctab = jnp.cos(fr)
    stab = jnp.sin(fr)

    full = lambda shp: pl.BlockSpec(shp, lambda *_: (0, 0))

    full3 = lambda shp: pl.BlockSpec(shp, lambda *_: (0, 0, 0))

    ki, kc3, vc3 = pl.pallas_call(
        _compress_kernel,
        out_shape=(jax.ShapeDtypeStruct((LCP, D), jnp.float32),
                   jax.ShapeDtypeStruct((H, LCP, HD), bf),
                   jax.ShapeDtypeStruct((H, LCP, HD), bf)),
        in_specs=[full((LCP, 4 * D)), full((LCP, 4 * D)), full((1, 8)),
                  full((D, D)), full((D, D)), full((D, D))],
        out_specs=(full((LCP, D)), full3((H, LCP, HD)),
                   full3((H, LCP, HD))),
    )(x4a, x4b, g2, Wi_k, Wk_c.astype(bf), Wv_c.astype(bf))

    bias = pl.pallas_call(
        _scores_kernel,
        grid=(NQB,),
        out_shape=jax.ShapeDtypeStruct((L, LCP), bf),
        in_specs=[pl.BlockSpec((QB, D), lambda i: (i, 0)),
                  pl.BlockSpec((D, D), lambda i: (0, 0)),
                  pl.BlockSpec((D, NIH), lambda i: (0, 0)),
                  pl.BlockSpec((LCP, D), lambda i: (0, 0))],
        out_specs=pl.BlockSpec((QB, LCP), lambda i: (i, 0)),
    )(x2, Wi_q, Wi_g, ki)

    xb16 = x2.astype(bf)
    sink3 = sink_logit.reshape(H, 1, 1)
    out = pl.pallas_call(
        _fused_kernel,
        grid=(NQB,),
        out_shape=jax.ShapeDtypeStruct((L, D), bf),
        in_specs=[
            pl.BlockSpec((QB, D), lambda i: (i, 0)),
            pl.BlockSpec((D, D), lambda i: (0, 0)),
            pl.BlockSpec((D, D), lambda i: (0, 0)),
            pl.BlockSpec((D, D), lambda i: (0, 0)),
            pl.BlockSpec((QB, 32), lambda i: (i, 0)),
            pl.BlockSpec((QB, 32), lambda i: (i, 0)),
            pl.BlockSpec((H, LCP, HD), lambda i: (0, 0, 0)),
            pl.BlockSpec((H, LCP, HD), lambda i: (0, 0, 0)),
            pl.BlockSpec((QB, LCP), lambda i: (i, 0)),
            pl.BlockSpec((H, 1, 1), lambda i: (0, 0, 0)),
            pl.BlockSpec((H, HD, D), lambda i: (0, 0, 0)),
        ],
        out_specs=pl.BlockSpec((QB, D), lambda i: (i, 0)),
        scratch_shapes=[pltpu.VMEM((QB, D), bf),
                        pltpu.VMEM((QB, D), bf)],
    )(xb16, Wq.astype(bf), Wk.astype(bf), Wv.astype(bf), ctab, stab,
      kc3, vc3, bias, sink3, Wo.astype(bf).reshape(H, HD, D))
    return out[None]
